# baseline (speedup 1.0000x reference)
import functools

import jax
import jax.numpy as jnp
from jax import lax
from jax.experimental import pallas as pl
from jax.experimental.pallas import tpu as pltpu
from jax.experimental.pallas import tpu_sc as plsc

N_E = 1024
WROWS = 304
E_DIM = 64
LEGACY = 0.2
B = 16
T = 576
ROWS = B * T
COLS_PAD = 1152
ROW_BLK = 1152
CHUNK = 64
CW = 256
TPAD = 640


def _dist_body(ks_ref, w_ref, p_ref, ab_ref, mi_ref):
    ks = ks_ref[...]
    w = w_ref[...]
    rown = jnp.sum(ks * ks, axis=1, keepdims=True)
    wn = jnp.sum(w * w, axis=1)
    col = lax.broadcasted_iota(jnp.int32, (1, COLS_PAD), 1)
    wn = jnp.where(col[0] <= N_E, wn, 1e9)
    dot = lax.dot_general(ks, w, (((1,), (1,)), ((), ())),
                          preferred_element_type=jnp.float32)
    d = (rown + wn[None, :]) - 2.0 * dot
    adv = (d[:, :N_E] > d[:, 1:N_E + 1]).astype(jnp.float32)
    pk = lax.dot_general(adv, p_ref[...], (((1,), (0,)), ((), ())),
                         preferred_element_type=jnp.float32)
    ab_ref[...] = pk.astype(jnp.int32)
    m = jnp.min(d, axis=1, keepdims=True)
    cols = lax.broadcasted_iota(jnp.int32, (ROW_BLK, COLS_PAD), 1)
    idx = jnp.min(jnp.where(d == m, cols, COLS_PAD), axis=1)
    mi_ref[...] = idx.reshape(1, ROW_BLK // 128, 128)


def _distances(ksf, w_pad, pmat):
    grid = ROWS // ROW_BLK
    return pl.pallas_call(
        _dist_body,
        grid=(grid,),
        in_specs=[
            pl.BlockSpec((ROW_BLK, E_DIM), lambda i: (i, 0)),
            pl.BlockSpec((COLS_PAD, E_DIM), lambda i: (0, 0)),
            pl.BlockSpec((N_E, E_DIM), lambda i: (0, 0)),
        ],
        out_specs=[
            pl.BlockSpec((ROW_BLK, E_DIM), lambda i: (i, 0)),
            pl.BlockSpec((1, ROW_BLK // 128, 128), lambda i: (i, 0, 0)),
        ],
        out_shape=[
            jax.ShapeDtypeStruct((ROWS, E_DIM), jnp.int32),
            jax.ShapeDtypeStruct((ROWS // ROW_BLK, ROW_BLK // 128, 128),
                                 jnp.int32),
        ],
    )(ksf, w_pad, pmat)


def _sc_body(d_hbm, mi_hbm, w_hbm, w64_hbm, enc_out, khh_out, khn_out,
             km_out, dbuf, wloc, kmbuf, encbuf, mibuf, sem):
    c = lax.axis_index("c")
    s = lax.axis_index("s")
    iota16 = lax.iota(jnp.int32, 16)
    lane0 = iota16 == 0
    zero16 = jnp.zeros((16,), jnp.int32)
    one16 = jnp.full((16,), 1, jnp.int32)
    cap = jnp.full((16,), N_E - 1, jnp.int32)

    @pl.when(s >= 8)
    def _km():
        b = c * 8 + (s - 8)
        pltpu.sync_copy(mi_hbm.at[b], mibuf)
        for j in range(4):
            for i in range(8):
                encbuf[pl.ds(j * 128 + i * 16, 16)] = \
                    mibuf[j, pl.ds(i * 16, 16)]
        for i in range(4):
            encbuf[pl.ds(512 + i * 16, 16)] = mibuf[4, pl.ds(i * 16, 16)]
        with jax.named_scope("km_gather"):
            for j in range(9):
                pltpu.async_copy(w_hbm.at[encbuf.at[pl.ds(j * 64, 64)]],
                                 kmbuf, sem).wait()
                pltpu.sync_copy(kmbuf,
                                km_out.at[pl.ds(b * T + j * 64, 64), :])

    @pl.when(s < 8)
    def _scan():
        b = c * 8 + s
        abits_cp = pltpu.make_async_copy(
            d_hbm.at[pl.ds(b * (T // 2), T // 2), :], dbuf, sem)
        abits_cp.start()
        pltpu.sync_copy(mi_hbm.at[b], mibuf)
        v0 = mibuf[0, pl.ds(0, 16)]
        enc0 = jnp.minimum(jnp.full((16,), v0[0], jnp.int32), cap)
        plsc.store_scatter(encbuf, [zero16], enc0, mask=lane0)
        base = (enc0[0] // 16) * 16
        win_cp = pltpu.make_async_copy(
            w64_hbm.at[pl.ds(pl.multiple_of(base // 2, 8), WROWS), :],
            wloc, sem)
        win_cp.start()
        with jax.named_scope("abits_wait"):
            abits_cp.wait()

        def stp(t, carry):
            ind, tl = carry
            col = ((tl & 1) << 6) + (ind >> 4)
            word = plsc.load_gather(dbuf, [tl >> 1, col])
            bit = (word >> (ind & 15)) & 1
            ind = jnp.minimum(ind + bit, cap)
            plsc.store_scatter(encbuf, [tl], ind, mask=lane0)
            return ind, tl + 1

        with jax.named_scope("scan_loop"):
            lax.fori_loop(1, T, stp, (enc0, one16))
        pltpu.sync_copy(encbuf, enc_out.at[pl.ds(b * T, T)])
        with jax.named_scope("win_wait"):
            win_cp.wait()
        basev = jnp.full((16,), base, jnp.int32)

        def expand(out_hbm, bump):
            def grp(g, carry):
                tv = jnp.full((16,), g * 16, jnp.int32) + iota16
                trow = tv >> 1
                tcol = (tv & 1) << 6
                ev = encbuf[pl.ds(g * 16, 16)]
                if bump:
                    ev = jnp.minimum(ev + 1, cap)
                rv = ev - basev
                prow = rv >> 1
                pcol = (rv & 1) << 6
                for col in range(E_DIM):
                    cv = jnp.full((16,), col, jnp.int32)
                    w = plsc.load_gather(wloc, [prow, pcol + cv])
                    plsc.store_scatter(dbuf, [trow, tcol + cv],
                                       plsc.bitcast(w, jnp.int32))
                return carry

            lax.fori_loop(0, T // 16, grp, 0)
            pltpu.sync_copy(dbuf, out_hbm.at[pl.ds(b * (T // 2), T // 2), :])

        with jax.named_scope("ex_khh"):
            expand(khh_out, False)
        with jax.named_scope("ex_khn"):
            expand(khn_out, True)


def _sc_scan_gather(d, mi_flat, w, w64):
    mesh = plsc.VectorSubcoreMesh(core_axis_name="c", subcore_axis_name="s")
    fn = pl.kernel(
        _sc_body,
        out_type=[
            jax.ShapeDtypeStruct((ROWS,), jnp.int32),
            jax.ShapeDtypeStruct((ROWS // 2, 128), jnp.int32),
            jax.ShapeDtypeStruct((ROWS // 2, 128), jnp.int32),
            jax.ShapeDtypeStruct((ROWS, 128), jnp.float32),
        ],
        mesh=mesh,
        compiler_params=pltpu.CompilerParams(needs_layout_passes=False),
        scratch_types=[
            pltpu.VMEM((T // 2, 128), jnp.int32),
            pltpu.VMEM((WROWS, 128), jnp.float32),
            pltpu.VMEM((64, 128), jnp.float32),
            pltpu.VMEM((T,), jnp.int32),
            pltpu.VMEM((TPAD // 128, 128), jnp.int32),
            pltpu.SemaphoreType.DMA,
        ],
    )
    return fn(d, mi_flat, w, w64)


def _loss_body(ks_ref, khh_ref, khn_ref, km_ref, enc_ref,
               kh_ref, lh_ref, ln_ref, em_ref, led_ref, v_ref):
    ks = ks_ref[...]
    khh = khh_ref[...]
    khn = khn_ref[...]
    km = km_ref[...][:, :E_DIM]

    def sq(a):
        r = ks - a
        return jnp.sum(r * r, axis=1).reshape(B, T)

    s1 = sq(khh)
    s2 = sq(khn)
    sm = sq(km)
    lhb = s1 + s1 * LEGACY
    lnb = s2 + s2 * LEGACY
    lmi = sm + sm * LEGACY
    energy = (s2 - s1) + (s2 - s1) * LEGACY
    lmh = jnp.where(lmi < lhb, lmi, 0.0)
    lmn = jnp.where(lmi < lnb, lmi, 0.0)
    lh_ref[...] = (lhb + (-lnb)) - lmh
    ln_ref[...] = (lnb + (-lhb)) - lmn

    enc = enc_ref[...]
    same = (enc[:, 1:] - enc[:, :-1]) == 0
    ec = (energy[:, 1:] - energy[:, :-1]) * jnp.where(same, 1.0, 0.0)
    led = jnp.maximum(ec + 1e-06 / N_E, 0.0)
    led_ref[0, 0] = jnp.sum(jnp.sum(led, axis=1)) / (B * (T - 1))
    em_ref[0, 0] = jnp.sum(jnp.sum(energy, axis=1)) / (B * T)
    mn = jnp.min(enc, axis=1)
    mx = jnp.max(enc, axis=1)
    v_ref[0, 0] = jnp.max(mx - mn)
    kh_ref[...] = ks + (khh - ks)


def _losses(ksf, khh, khn, km, enc):
    return pl.pallas_call(
        _loss_body,
        out_specs=[
            pl.BlockSpec((ROWS, E_DIM), lambda: (0, 0)),
            pl.BlockSpec((B, T), lambda: (0, 0)),
            pl.BlockSpec((B, T), lambda: (0, 0)),
            pl.BlockSpec(memory_space=pltpu.SMEM),
            pl.BlockSpec(memory_space=pltpu.SMEM),
            pl.BlockSpec(memory_space=pltpu.SMEM),
        ],
        out_shape=[
            jax.ShapeDtypeStruct((ROWS, E_DIM), jnp.float32),
            jax.ShapeDtypeStruct((B, T), jnp.float32),
            jax.ShapeDtypeStruct((B, T), jnp.float32),
            jax.ShapeDtypeStruct((1, 1), jnp.float32),
            jax.ShapeDtypeStruct((1, 1), jnp.float32),
            jax.ShapeDtypeStruct((1, 1), jnp.int32),
        ],
    )(ksf, khh, khn, km, enc)


def kernel(key_soft, W):
    ksf = key_soft.reshape(ROWS, E_DIM)
    w_pad = jnp.zeros((COLS_PAD, E_DIM), jnp.float32).at[: N_E + 1].set(W)
    j = jnp.arange(N_E)
    pmat = jnp.where((j[:, None] // 16) == jnp.arange(E_DIM)[None, :],
                     2.0 ** (j[:, None] % 16), 0.0).astype(jnp.float32)
    abits, mi = _distances(ksf, w_pad, pmat)
    mi3 = jnp.pad(mi.reshape(B, T), ((0, 0), (0, TPAD - T))).reshape(
        B, TPAD // 128, 128)
    w_g = jnp.pad(W, ((0, 0), (0, 128 - E_DIM)))
    w64 = jnp.pad(W, ((0, 2 * 832 - 1025), (0, 0))).reshape(832, 128)
    ab2 = abits.reshape(ROWS // 2, 128)
    enc_flat, khh_i, khn_i, km = _sc_scan_gather(ab2, mi3, w_g, w64)
    khh = lax.bitcast_convert_type(khh_i.reshape(ROWS, E_DIM), jnp.float32)
    khn = lax.bitcast_convert_type(khn_i.reshape(ROWS, E_DIM), jnp.float32)
    enc = enc_flat.reshape(B, T)
    key_hard, loss_here, loss_next, em, led, v = _losses(
        ksf, khh, khn, km, enc)
    return (key_hard.reshape(key_soft.shape), enc, v[0, 0],
            loss_here, loss_next, em[0, 0], led[0, 0])


# speedup vs baseline: 1.2226x; 1.2226x over previous
import functools

import jax
import jax.numpy as jnp
from jax import lax
from jax.experimental import pallas as pl
from jax.experimental.pallas import tpu as pltpu
from jax.experimental.pallas import tpu_sc as plsc

N_E = 1024
WROWS = 304
E_DIM = 64
LEGACY = 0.2
B = 16
T = 576
ROWS = B * T
COLS_PAD = 1152
ROW_BLK = 1152
CHUNK = 64
CW = 256
TPAD = 640


def _dist_body(ks_ref, w_ref, p_ref, ab_ref, mi_ref):
    ks = ks_ref[...]
    w = w_ref[...]
    rown = jnp.sum(ks * ks, axis=1, keepdims=True)
    wn = jnp.sum(w * w, axis=1)
    col = lax.broadcasted_iota(jnp.int32, (1, COLS_PAD), 1)
    wn = jnp.where(col[0] <= N_E, wn, 1e9)
    dot = lax.dot_general(ks, w, (((1,), (1,)), ((), ())),
                          preferred_element_type=jnp.float32)
    d = (rown + wn[None, :]) - 2.0 * dot
    adv = (d[:, :N_E] > d[:, 1:N_E + 1]).astype(jnp.float32)
    pk = lax.dot_general(adv, p_ref[...], (((1,), (0,)), ((), ())),
                         preferred_element_type=jnp.float32)
    ab_ref[...] = pk.astype(jnp.int32)
    m = jnp.min(d, axis=1, keepdims=True)
    cols = lax.broadcasted_iota(jnp.int32, (ROW_BLK, COLS_PAD), 1)
    idx = jnp.min(jnp.where(d == m, cols, COLS_PAD), axis=1)
    mi_ref[...] = idx.reshape(1, ROW_BLK // 128, 128)


def _distances(ksf, w_pad, pmat):
    grid = ROWS // ROW_BLK
    return pl.pallas_call(
        _dist_body,
        grid=(grid,),
        in_specs=[
            pl.BlockSpec((ROW_BLK, E_DIM), lambda i: (i, 0)),
            pl.BlockSpec((COLS_PAD, E_DIM), lambda i: (0, 0)),
            pl.BlockSpec((N_E, E_DIM), lambda i: (0, 0)),
        ],
        out_specs=[
            pl.BlockSpec((ROW_BLK, E_DIM), lambda i: (i, 0)),
            pl.BlockSpec((1, ROW_BLK // 128, 128), lambda i: (i, 0, 0)),
        ],
        out_shape=[
            jax.ShapeDtypeStruct((ROWS, E_DIM), jnp.int32),
            jax.ShapeDtypeStruct((ROWS // ROW_BLK, ROW_BLK // 128, 128),
                                 jnp.int32),
        ],
    )(ksf, w_pad, pmat)


def _sc_body(d_hbm, mi_hbm, w_hbm, w64_hbm, enc_out, khh_out, khn_out,
             km_out, dbuf, wloc, kmbuf, encbuf, mibuf, sem):
    c = lax.axis_index("c")
    s = lax.axis_index("s")
    iota16 = lax.iota(jnp.int32, 16)
    lane0 = iota16 == 0
    zero16 = jnp.zeros((16,), jnp.int32)
    one16 = jnp.full((16,), 1, jnp.int32)
    cap = jnp.full((16,), N_E - 1, jnp.int32)

    @pl.when(s >= 8)
    def _km():
        b = c * 8 + (s - 8)
        pltpu.sync_copy(mi_hbm.at[b], mibuf)
        for j in range(4):
            for i in range(8):
                encbuf[pl.ds(j * 128 + i * 16, 16)] = \
                    mibuf[j, pl.ds(i * 16, 16)]
        for i in range(4):
            encbuf[pl.ds(512 + i * 16, 16)] = mibuf[4, pl.ds(i * 16, 16)]
        with jax.named_scope("km_gather"):
            for j in range(9):
                pltpu.async_copy(w_hbm.at[encbuf.at[pl.ds(j * 64, 64)]],
                                 kmbuf, sem).wait()
                pltpu.sync_copy(kmbuf,
                                km_out.at[pl.ds(b * T + j * 64, 64), :])

    @pl.when(s < 8)
    def _scan():
        b = c * 8 + s
        abits_cp = pltpu.make_async_copy(
            d_hbm.at[pl.ds(b * (T // 2), T // 2), :], dbuf, sem)
        abits_cp.start()
        pltpu.sync_copy(mi_hbm.at[b], mibuf)
        v0 = mibuf[0, pl.ds(0, 16)]
        enc0 = jnp.minimum(jnp.full((16,), v0[0], jnp.int32), cap)
        plsc.store_scatter(encbuf, [zero16], enc0, mask=lane0)
        base = (enc0[0] // 16) * 16
        win_cp = pltpu.make_async_copy(
            w64_hbm.at[pl.ds(pl.multiple_of(base // 2, 8), WROWS), :],
            wloc, sem)
        win_cp.start()
        with jax.named_scope("abits_wait"):
            abits_cp.wait()

        def stp(t, carry):
            ind, tl = carry
            col = ((tl & 1) << 6) + (ind >> 4)
            word = plsc.load_gather(dbuf, [tl >> 1, col])
            bit = (word >> (ind & 15)) & 1
            ind = jnp.minimum(ind + bit, cap)
            plsc.store_scatter(encbuf, [tl], ind, mask=lane0)
            return ind, tl + 1

        with jax.named_scope("scan_loop"):
            lax.fori_loop(1, T, stp, (enc0, one16))
        pltpu.sync_copy(encbuf, enc_out.at[pl.ds(b * T, T)])
        with jax.named_scope("win_wait"):
            win_cp.wait()
        basev = jnp.full((16,), base, jnp.int32)

        def expand(out_hbm, bump):
            @plsc.parallel_loop(0, T // 16, 1, unroll=2)
            def grp(g):
                tv = jnp.full((16,), g * 16, jnp.int32) + iota16
                trow = tv >> 1
                tcol = (tv & 1) << 6
                ev = encbuf[pl.ds(g * 16, 16)]
                if bump:
                    ev = jnp.minimum(ev + 1, cap)
                rv = ev - basev
                prow = rv >> 1
                pcol = (rv & 1) << 6
                for col in range(E_DIM):
                    cv = jnp.full((16,), col, jnp.int32)
                    w = plsc.load_gather(wloc, [prow, pcol + cv])
                    plsc.store_scatter(dbuf, [trow, tcol + cv],
                                       plsc.bitcast(w, jnp.int32))
            pltpu.sync_copy(dbuf, out_hbm.at[pl.ds(b * (T // 2), T // 2), :])

        with jax.named_scope("ex_khh"):
            expand(khh_out, False)
        with jax.named_scope("ex_khn"):
            expand(khn_out, True)


def _sc_scan_gather(d, mi_flat, w, w64):
    mesh = plsc.VectorSubcoreMesh(core_axis_name="c", subcore_axis_name="s")
    fn = pl.kernel(
        _sc_body,
        out_type=[
            jax.ShapeDtypeStruct((ROWS,), jnp.int32),
            jax.ShapeDtypeStruct((ROWS // 2, 128), jnp.int32),
            jax.ShapeDtypeStruct((ROWS // 2, 128), jnp.int32),
            jax.ShapeDtypeStruct((ROWS, 128), jnp.float32),
        ],
        mesh=mesh,
        compiler_params=pltpu.CompilerParams(needs_layout_passes=False),
        scratch_types=[
            pltpu.VMEM((T // 2, 128), jnp.int32),
            pltpu.VMEM((WROWS, 128), jnp.float32),
            pltpu.VMEM((64, 128), jnp.float32),
            pltpu.VMEM((T,), jnp.int32),
            pltpu.VMEM((TPAD // 128, 128), jnp.int32),
            pltpu.SemaphoreType.DMA,
        ],
    )
    return fn(d, mi_flat, w, w64)


def _loss_body(ks_ref, khh_ref, khn_ref, km_ref, enc_ref,
               kh_ref, lh_ref, ln_ref, em_ref, led_ref, v_ref):
    ks = ks_ref[...]
    khh = khh_ref[...]
    khn = khn_ref[...]
    km = km_ref[...][:, :E_DIM]

    def sq(a):
        r = ks - a
        return jnp.sum(r * r, axis=1).reshape(B, T)

    s1 = sq(khh)
    s2 = sq(khn)
    sm = sq(km)
    lhb = s1 + s1 * LEGACY
    lnb = s2 + s2 * LEGACY
    lmi = sm + sm * LEGACY
    energy = (s2 - s1) + (s2 - s1) * LEGACY
    lmh = jnp.where(lmi < lhb, lmi, 0.0)
    lmn = jnp.where(lmi < lnb, lmi, 0.0)
    lh_ref[...] = (lhb + (-lnb)) - lmh
    ln_ref[...] = (lnb + (-lhb)) - lmn

    enc = enc_ref[...]
    same = (enc[:, 1:] - enc[:, :-1]) == 0
    ec = (energy[:, 1:] - energy[:, :-1]) * jnp.where(same, 1.0, 0.0)
    led = jnp.maximum(ec + 1e-06 / N_E, 0.0)
    led_ref[0, 0] = jnp.sum(jnp.sum(led, axis=1)) / (B * (T - 1))
    em_ref[0, 0] = jnp.sum(jnp.sum(energy, axis=1)) / (B * T)
    mn = jnp.min(enc, axis=1)
    mx = jnp.max(enc, axis=1)
    v_ref[0, 0] = jnp.max(mx - mn)
    kh_ref[...] = ks + (khh - ks)


def _losses(ksf, khh, khn, km, enc):
    return pl.pallas_call(
        _loss_body,
        out_specs=[
            pl.BlockSpec((ROWS, E_DIM), lambda: (0, 0)),
            pl.BlockSpec((B, T), lambda: (0, 0)),
            pl.BlockSpec((B, T), lambda: (0, 0)),
            pl.BlockSpec(memory_space=pltpu.SMEM),
            pl.BlockSpec(memory_space=pltpu.SMEM),
            pl.BlockSpec(memory_space=pltpu.SMEM),
        ],
        out_shape=[
            jax.ShapeDtypeStruct((ROWS, E_DIM), jnp.float32),
            jax.ShapeDtypeStruct((B, T), jnp.float32),
            jax.ShapeDtypeStruct((B, T), jnp.float32),
            jax.ShapeDtypeStruct((1, 1), jnp.float32),
            jax.ShapeDtypeStruct((1, 1), jnp.float32),
            jax.ShapeDtypeStruct((1, 1), jnp.int32),
        ],
    )(ksf, khh, khn, km, enc)


def kernel(key_soft, W):
    ksf = key_soft.reshape(ROWS, E_DIM)
    w_pad = jnp.zeros((COLS_PAD, E_DIM), jnp.float32).at[: N_E + 1].set(W)
    j = jnp.arange(N_E)
    pmat = jnp.where((j[:, None] // 16) == jnp.arange(E_DIM)[None, :],
                     2.0 ** (j[:, None] % 16), 0.0).astype(jnp.float32)
    abits, mi = _distances(ksf, w_pad, pmat)
    mi3 = jnp.pad(mi.reshape(B, T), ((0, 0), (0, TPAD - T))).reshape(
        B, TPAD // 128, 128)
    w_g = jnp.pad(W, ((0, 0), (0, 128 - E_DIM)))
    w64 = jnp.pad(W, ((0, 2 * 832 - 1025), (0, 0))).reshape(832, 128)
    ab2 = abits.reshape(ROWS // 2, 128)
    enc_flat, khh_i, khn_i, km = _sc_scan_gather(ab2, mi3, w_g, w64)
    khh = lax.bitcast_convert_type(khh_i.reshape(ROWS, E_DIM), jnp.float32)
    khn = lax.bitcast_convert_type(khn_i.reshape(ROWS, E_DIM), jnp.float32)
    enc = enc_flat.reshape(B, T)
    key_hard, loss_here, loss_next, em, led, v = _losses(
        ksf, khh, khn, km, enc)
    return (key_hard.reshape(key_soft.shape), enc, v[0, 0],
            loss_here, loss_next, em[0, 0], led[0, 0])
